# Initial kernel scaffold; baseline (speedup 1.0000x reference)
import jax
import jax.numpy as jnp
from jax.experimental import pallas as pl


def kernel(features):
    raise NotImplementedError("write your pallas kernel here")


# speedup vs baseline: 1.3650x; 1.3650x over previous
import functools

import jax
import jax.numpy as jnp
from jax import lax
from jax.experimental import pallas as pl
from jax.experimental.pallas import tpu as pltpu
from jax.experimental.pallas import tpu_sc as plsc

N = 100000
C = 256
CO = C // 2
NC, NS = 2, 16
NW = NC * NS
TILE_ROWS = N // NW
R = 125
CHUNKS = TILE_ROWS // R
IN_CHUNK = R * C
OUT_CHUNK = R * CO
VECS = OUT_CHUNK // 16


def _sc_body(feat_ref, out_ref, in0, in1, ot0, ot1, si0, si1, so0, so1):
    wid = lax.axis_index("s") * NC + lax.axis_index("c")
    in_base = wid * (TILE_ROWS * C)
    out_base = wid * (TILE_ROWS * CO)
    iota2 = lax.iota(jnp.int32, 16) * 2

    in_b = (in0, in1)
    out_b = (ot0, ot1)
    si = (si0, si1)
    so = (so0, so1)

    def start_in(c):
        b = c % 2
        return pltpu.async_copy(
            feat_ref.at[pl.ds(in_base + c * IN_CHUNK, IN_CHUNK)], in_b[b], si[b]
        )

    in_h = [start_in(0), None]
    out_h = [None, None]
    for c in range(CHUNKS):
        b = c % 2
        if c + 1 < CHUNKS:
            in_h[(c + 1) % 2] = start_in(c + 1)
        in_h[b].wait()
        if out_h[b] is not None:
            out_h[b].wait()
        src = in_b[b]
        dst = out_b[b]

        @plsc.parallel_loop(0, VECS, 1, unroll=8)
        def _chunk(k):
            dst[pl.ds(k * 16, 16)] = plsc.load_gather(src, [k * 32 + iota2])

        out_h[b] = pltpu.async_copy(
            dst, out_ref.at[pl.ds(out_base + c * OUT_CHUNK, OUT_CHUNK)], so[b]
        )
    for b in (0, 1):
        if out_h[b] is not None:
            out_h[b].wait()


@jax.jit
def kernel(features):
    flat = features.reshape(-1)
    run = pl.kernel(
        _sc_body,
        out_type=jax.ShapeDtypeStruct((N * CO,), jnp.float32),
        mesh=plsc.VectorSubcoreMesh(core_axis_name="c", subcore_axis_name="s"),
        compiler_params=pltpu.CompilerParams(needs_layout_passes=False),
        scratch_types=[
            pltpu.VMEM((IN_CHUNK,), jnp.float32),
            pltpu.VMEM((IN_CHUNK,), jnp.float32),
            pltpu.VMEM((OUT_CHUNK,), jnp.float32),
            pltpu.VMEM((OUT_CHUNK,), jnp.float32),
            pltpu.SemaphoreType.DMA,
            pltpu.SemaphoreType.DMA,
            pltpu.SemaphoreType.DMA,
            pltpu.SemaphoreType.DMA,
        ],
    )
    return run(flat).reshape(N, CO)


# speedup vs baseline: 1.3765x; 1.0084x over previous
import functools

import jax
import jax.numpy as jnp
from jax import lax
from jax.experimental import pallas as pl
from jax.experimental.pallas import tpu as pltpu
from jax.experimental.pallas import tpu_sc as plsc

N = 100000
C = 256
CO = C // 2
NC, NS = 2, 16
NW = NC * NS
TILE_ROWS = N // NW
R = 125
CHUNKS = TILE_ROWS // R
IN_CHUNK = R * C
OUT_CHUNK = R * CO
VECS = OUT_CHUNK // 16


def _sc_body(feat_ref, out_ref, in0, in1, ot0, ot1, si0, si1, so0, so1):
    wid = lax.axis_index("s") * NC + lax.axis_index("c")
    in_base = wid * (TILE_ROWS * C)
    out_base = wid * (TILE_ROWS * CO)
    iota2 = lax.iota(jnp.int32, 16) * 2

    in_b = (in0, in1)
    out_b = (ot0, ot1)
    si = (si0, si1)
    so = (so0, so1)

    def start_in(c):
        b = c % 2
        return pltpu.async_copy(
            feat_ref.at[pl.ds(in_base + c * IN_CHUNK, IN_CHUNK)], in_b[b], si[b]
        )

    in_h = [start_in(0), None]
    out_h = [None, None]
    for c in range(CHUNKS):
        b = c % 2
        if c + 1 < CHUNKS:
            in_h[(c + 1) % 2] = start_in(c + 1)
        in_h[b].wait()
        if out_h[b] is not None:
            out_h[b].wait()
        src = in_b[b]
        dst = out_b[b]

        @plsc.parallel_loop(0, VECS, 1, unroll=8, carry=iota2)
        def _chunk(k, idx):
            dst[pl.ds(k * 16, 16)] = plsc.load_gather(src, [idx])
            return idx + 32

        out_h[b] = pltpu.async_copy(
            dst, out_ref.at[pl.ds(out_base + c * OUT_CHUNK, OUT_CHUNK)], so[b]
        )
    for b in (0, 1):
        if out_h[b] is not None:
            out_h[b].wait()


@jax.jit
def kernel(features):
    flat = features.reshape(-1)
    run = pl.kernel(
        _sc_body,
        out_type=jax.ShapeDtypeStruct((N * CO,), jnp.float32),
        mesh=plsc.VectorSubcoreMesh(core_axis_name="c", subcore_axis_name="s"),
        compiler_params=pltpu.CompilerParams(needs_layout_passes=False),
        scratch_types=[
            pltpu.VMEM((IN_CHUNK,), jnp.float32),
            pltpu.VMEM((IN_CHUNK,), jnp.float32),
            pltpu.VMEM((OUT_CHUNK,), jnp.float32),
            pltpu.VMEM((OUT_CHUNK,), jnp.float32),
            pltpu.SemaphoreType.DMA,
            pltpu.SemaphoreType.DMA,
            pltpu.SemaphoreType.DMA,
            pltpu.SemaphoreType.DMA,
        ],
    )
    return run(flat).reshape(N, CO)
